# Initial kernel scaffold; baseline (speedup 1.0000x reference)
import jax
import jax.numpy as jnp
from jax.experimental import pallas as pl


def kernel(x):
    raise NotImplementedError("write your pallas kernel here")


# speedup vs baseline: 34.9856x; 34.9856x over previous
import functools

import jax
import jax.numpy as jnp
from jax.experimental import pallas as pl

_NEWTON_ITERS = 16


def _sparsemax_block(x_ref, o_ref):
    x = x_ref[...]
    tau = jnp.max(x, axis=-1, keepdims=True) - 1.0

    def body(_, tau):
        mask = x > tau
        s = jnp.sum(jnp.where(mask, x, 0.0), axis=-1, keepdims=True)
        c = jnp.sum(mask.astype(jnp.float32), axis=-1, keepdims=True)
        return (s - 1.0) / c

    tau = jax.lax.fori_loop(0, _NEWTON_ITERS, body, tau, unroll=True)
    o_ref[...] = jnp.maximum(x - tau, 0.0)


@functools.partial(jax.jit, static_argnames=())
def kernel(x):
    return pl.pallas_call(
        _sparsemax_block,
        out_shape=jax.ShapeDtypeStruct(x.shape, x.dtype),
    )(x)


# speedup vs baseline: 51.5291x; 1.4729x over previous
import functools

import jax
import jax.numpy as jnp
from jax.experimental import pallas as pl

_UNROLLED_ITERS = 6
_MAX_EXTRA_ITERS = 26


def _sparsemax_block(x_ref, o_ref):
    x = x_ref[...]

    def newton(tau):
        mask = x > tau
        g = jnp.where(mask, x - tau, 0.0)
        s = jnp.sum(g, axis=-1, keepdims=True)
        c = jnp.sum(mask.astype(jnp.float32), axis=-1, keepdims=True)
        return tau + (s - 1.0) / c

    tau = jnp.max(x, axis=-1, keepdims=True) - 1.0
    for _ in range(_UNROLLED_ITERS):
        tau = newton(tau)

    def cond(carry):
        k, _, changed = carry
        return jnp.logical_and(k < _MAX_EXTRA_ITERS, changed)

    def body(carry):
        k, tau, _ = carry
        tau_new = newton(tau)
        return k + 1, tau_new, jnp.any(tau_new != tau)

    _, tau, _ = jax.lax.while_loop(cond, body, (0, tau, jnp.bool_(True)))
    o_ref[...] = jnp.maximum(x - tau, 0.0)


@functools.partial(jax.jit, static_argnames=())
def kernel(x):
    return pl.pallas_call(
        _sparsemax_block,
        out_shape=jax.ShapeDtypeStruct(x.shape, x.dtype),
    )(x)
